# baseline (speedup 1.0000x reference)
import functools

import jax
import jax.numpy as jnp
from jax import lax
from jax.experimental import pallas as pl
from jax.experimental.pallas import tpu as pltpu
from jax.experimental.pallas import tpu_sc as plsc

_B = 16384
_EMB = 32
_CH = 128


def _sc_gather(user_id, product_id, user_table, product_table):
    info = plsc.get_sparse_core_info()
    nw = info.num_cores * info.num_subcores
    b_per_w = _B // nw
    nch = b_per_w // _CH
    mesh = plsc.VectorSubcoreMesh(core_axis_name="c", subcore_axis_name="s")

    @functools.partial(
        pl.kernel,
        mesh=mesh,
        compiler_params=pltpu.CompilerParams(use_tc_tiling_on_sc=False),
        out_type=(
            jax.ShapeDtypeStruct((_B, _EMB), jnp.float32),
            jax.ShapeDtypeStruct((_B, _EMB), jnp.float32),
        ),
        scratch_types=[
            pltpu.VMEM((b_per_w,), jnp.int32),
            pltpu.VMEM((b_per_w,), jnp.int32),
            pltpu.VMEM((b_per_w, _EMB), jnp.float32),
            pltpu.VMEM((b_per_w, _EMB), jnp.float32),
            pltpu.SemaphoreType.DMA,
        ],
    )
    def gather_k(uid_hbm, pid_hbm, utab_hbm, ptab_hbm, uout_hbm, pout_hbm,
                 uidx_v, pidx_v, urows_v, prows_v, sem):
        wid = lax.axis_index("s") * info.num_cores + lax.axis_index("c")
        base = wid * b_per_w
        pltpu.sync_copy(uid_hbm.at[pl.ds(base, b_per_w)], uidx_v)
        pltpu.sync_copy(pid_hbm.at[pl.ds(base, b_per_w)], pidx_v)
        copies = []
        for j in range(nch):
            sl = pl.ds(j * _CH, _CH)
            copies.append(
                pltpu.async_copy(utab_hbm.at[uidx_v.at[sl]], urows_v.at[sl], sem))
            copies.append(
                pltpu.async_copy(ptab_hbm.at[pidx_v.at[sl]], prows_v.at[sl], sem))
        for c in copies:
            c.wait()
        pltpu.sync_copy(urows_v, uout_hbm.at[pl.ds(base, b_per_w)])
        pltpu.sync_copy(prows_v, pout_hbm.at[pl.ds(base, b_per_w)])

    return gather_k(user_id, product_id, user_table, product_table)


def _mlp_body(u_ref, p_ref, w1u_ref, w1p_ref, b1_ref, w2_ref, b2_ref,
              w3_ref, b3_ref, out_ref):
    h = u_ref[...] @ w1u_ref[...] + p_ref[...] @ w1p_ref[...] + b1_ref[...]
    h = jnp.maximum(h, 0.0)
    h = jnp.maximum(h @ w2_ref[...] + b2_ref[...], 0.0)
    out_ref[...] = h @ w3_ref[...] + b3_ref[...]


def _mlp(u_emb, p_emb, W1, b1, W2, b2, W3, b3):
    bb = 2048
    grid = (_B // bb,)
    return pl.pallas_call(
        _mlp_body,
        grid=grid,
        in_specs=[
            pl.BlockSpec((bb, _EMB), lambda i: (i, 0)),
            pl.BlockSpec((bb, _EMB), lambda i: (i, 0)),
            pl.BlockSpec((_EMB, 256), lambda i: (0, 0)),
            pl.BlockSpec((_EMB, 256), lambda i: (0, 0)),
            pl.BlockSpec((1, 256), lambda i: (0, 0)),
            pl.BlockSpec((256, 128), lambda i: (0, 0)),
            pl.BlockSpec((1, 128), lambda i: (0, 0)),
            pl.BlockSpec((128, 1), lambda i: (0, 0)),
            pl.BlockSpec((1, 1), lambda i: (0, 0)),
        ],
        out_specs=pl.BlockSpec((bb, 1), lambda i: (i, 0)),
        out_shape=jax.ShapeDtypeStruct((_B, 1), jnp.float32),
    )(u_emb, p_emb, W1[:_EMB], W1[_EMB:], b1.reshape(1, 256), W2,
      b2.reshape(1, 128), W3, b3.reshape(1, 1))


def kernel(user_id, product_id, user_table, product_table,
           W1, b1, W2, b2, W3, b3):
    u_emb, p_emb = _sc_gather(user_id, product_id, user_table, product_table)
    rating = _mlp(u_emb, p_emb, W1, b1, W2, b2, W3, b3)
    return (u_emb, p_emb, rating)
